# Initial kernel scaffold; baseline (speedup 1.0000x reference)
import jax
import jax.numpy as jnp
from jax.experimental import pallas as pl


def kernel(x, table):
    raise NotImplementedError("write your pallas kernel here")


# speedup vs baseline: 1.1104x; 1.1104x over previous
import functools

import jax
import jax.numpy as jnp
from jax import lax
from jax.experimental import pallas as pl
from jax.experimental.pallas import tpu as pltpu
from jax.experimental.pallas import tpu_sc as plsc

NUM_ROWS = 16384
SEQ = 50
DIM = 32
B = NUM_ROWS * SEQ
NC, NS = 2, 16
NW = NC * NS
B_PER_W = B // NW
CHUNK = 3200
NCHUNK = B_PER_W // CHUNK

_mesh = plsc.VectorSubcoreMesh(core_axis_name="c", subcore_axis_name="s")


@functools.partial(
    pl.kernel,
    mesh=_mesh,
    out_type=jax.ShapeDtypeStruct((B, DIM), jnp.float32),
    scratch_types=[
        pltpu.VMEM((CHUNK,), jnp.int32),
        pltpu.VMEM((CHUNK, DIM), jnp.float32),
        pltpu.SemaphoreType.DMA,
    ],
    compiler_params=pltpu.CompilerParams(use_tc_tiling_on_sc=False),
)
def _gather_kernel(x_hbm, table_hbm, out_hbm, idx_v, rows_v, sem):
    wid = lax.axis_index("s") * NC + lax.axis_index("c")
    base = wid * B_PER_W
    for g in range(NCHUNK):
        off = base + g * CHUNK
        pltpu.sync_copy(x_hbm.at[pl.ds(off, CHUNK)], idx_v)
        pltpu.async_copy(table_hbm.at[idx_v], rows_v, sem).wait()
        pltpu.sync_copy(rows_v, out_hbm.at[pl.ds(off, CHUNK)])


def kernel(x, table):
    out = _gather_kernel(x.reshape(B), table)
    return out.reshape(NUM_ROWS, SEQ, DIM)


# speedup vs baseline: 1.1122x; 1.0016x over previous
import functools

import jax
import jax.numpy as jnp
from jax import lax
from jax.experimental import pallas as pl
from jax.experimental.pallas import tpu as pltpu
from jax.experimental.pallas import tpu_sc as plsc

NUM_ROWS = 16384
SEQ = 50
DIM = 32
B = NUM_ROWS * SEQ
NC, NS = 2, 16
NW = NC * NS
B_PER_W = B // NW
CHUNK = 1024
NCHUNK = B_PER_W // CHUNK
NBUF = 3

_mesh = plsc.VectorSubcoreMesh(core_axis_name="c", subcore_axis_name="s")


@functools.partial(
    pl.kernel,
    mesh=_mesh,
    out_type=jax.ShapeDtypeStruct((B, DIM), jnp.float32),
    scratch_types=[
        pltpu.VMEM((B_PER_W,), jnp.int32),
        [pltpu.VMEM((CHUNK, DIM), jnp.float32) for _ in range(NBUF)],
        [pltpu.SemaphoreType.DMA for _ in range(NBUF)],
        [pltpu.SemaphoreType.DMA for _ in range(NBUF)],
    ],
    compiler_params=pltpu.CompilerParams(use_tc_tiling_on_sc=False),
)
def _gather_kernel(x_hbm, table_hbm, out_hbm, idx_v, rows, gsem, ssem):
    wid = lax.axis_index("s") * NC + lax.axis_index("c")
    base = wid * B_PER_W
    pltpu.sync_copy(x_hbm.at[pl.ds(base, B_PER_W)], idx_v)

    gathers, stores = {}, {}

    def issue_gather(g):
        b = g % NBUF
        gathers[g] = pltpu.async_copy(
            table_hbm.at[idx_v.at[pl.ds(g * CHUNK, CHUNK)]], rows[b], gsem[b]
        )

    def issue_store(g):
        b = g % NBUF
        stores[g] = pltpu.async_copy(
            rows[b], out_hbm.at[pl.ds(base + g * CHUNK, CHUNK)], ssem[b]
        )

    for j in range(NBUF - 1):
        issue_gather(j)
    for g in range(NCHUNK):
        nxt = g + NBUF - 1
        if nxt < NCHUNK:
            if nxt >= NBUF:
                stores[nxt - NBUF].wait()
            issue_gather(nxt)
        gathers[g].wait()
        issue_store(g)
    for g in range(NCHUNK - NBUF, NCHUNK):
        stores[g].wait()


def kernel(x, table):
    out = _gather_kernel(x.reshape(B), table)
    return out.reshape(NUM_ROWS, SEQ, DIM)


# speedup vs baseline: 1.8084x; 1.6260x over previous
import functools

import jax
import jax.numpy as jnp
from jax import lax
from jax.experimental import pallas as pl
from jax.experimental.pallas import tpu as pltpu
from jax.experimental.pallas import tpu_sc as plsc

NUM_ROWS = 16384
SEQ = 50
DIM = 32
B = NUM_ROWS * SEQ
NC, NS = 2, 16
NW = NC * NS
R_PER_W = NUM_ROWS // NW
B_PER_W = R_PER_W * SEQ
CHUNK_R = 16
CHUNK = CHUNK_R * SEQ
NCHUNK = R_PER_W // CHUNK_R
NBUF = 2
NROUND = NCHUNK // NBUF

_mesh = plsc.VectorSubcoreMesh(core_axis_name="c", subcore_axis_name="s")


@functools.partial(
    pl.kernel,
    mesh=_mesh,
    out_type=jax.ShapeDtypeStruct((NUM_ROWS, SEQ, DIM), jnp.float32),
    scratch_types=[
        pltpu.VMEM((B_PER_W,), jnp.int32),
        [pltpu.VMEM((CHUNK, DIM), jnp.float32) for _ in range(NBUF)],
        [pltpu.SemaphoreType.DMA for _ in range(NBUF)],
        [pltpu.SemaphoreType.DMA for _ in range(NBUF)],
    ],
    compiler_params=pltpu.CompilerParams(use_tc_tiling_on_sc=False),
)
def _gather_kernel(x_hbm, table_hbm, out_hbm, idx_v, rows, gsem, ssem):
    wid = lax.axis_index("s") * NC + lax.axis_index("c")
    base = wid * B_PER_W
    row0 = wid * R_PER_W
    pltpu.sync_copy(x_hbm.at[pl.ds(base, B_PER_W)], idx_v)

    def idx_slice(gc):
        return idx_v.at[pl.ds(pl.multiple_of(gc * CHUNK, 8), CHUNK)]

    def gather_copy(gc, b):
        return pltpu.make_async_copy(
            table_hbm.at[idx_slice(gc)], rows[b], gsem[b]
        )

    for b in range(NBUF):
        gather_copy(b, b).start()

    def round_body(r, carry):
        for b in range(NBUF):
            gc = r * NBUF + b
            gather_copy(gc, b).wait()
            for j in range(CHUNK_R):
                pltpu.make_async_copy(
                    rows[b].at[pl.ds(j * SEQ, SEQ)],
                    out_hbm.at[row0 + gc * CHUNK_R + j],
                    ssem[b],
                ).start()
            for j in range(CHUNK_R):
                pltpu.make_async_copy(
                    rows[b].at[pl.ds(j * SEQ, SEQ)],
                    out_hbm.at[row0 + gc * CHUNK_R + j],
                    ssem[b],
                ).wait()

            @pl.when(gc + NBUF < NCHUNK)
            def _():
                gather_copy(gc + NBUF, b).start()

        return carry

    lax.fori_loop(0, NROUND, round_body, 0)


def kernel(x, table):
    return _gather_kernel(x.reshape(B), table)
